# Initial kernel scaffold; baseline (speedup 1.0000x reference)
import jax
import jax.numpy as jnp
from jax.experimental import pallas as pl


def kernel(z, codebook, var_q, var_init):
    raise NotImplementedError("write your pallas kernel here")


# speedup vs baseline: 2.1965x; 2.1965x over previous
import functools

import jax
import jax.numpy as jnp
from jax import lax
from jax.experimental import pallas as pl
from jax.experimental.pallas import tpu as pltpu

PB = 512


def _vq_body(w_ref, z_ref, cb_ref, zq_ref, loss_ref, *, nb, size, inv_bs):
    b = pl.program_id(0)
    p = pl.program_id(1)
    w = w_ref[0, 0]
    zb = z_ref[0]
    cb = cb_ref[...]

    m = lax.dot_general(zb, cb, (((0,), (1,)), ((), ())),
                        preferred_element_type=jnp.float32)
    cbsq = jnp.sum(cb * cb, axis=1)
    g = (2.0 * w) * m - w * cbsq[None, :]

    rowmax = jnp.max(g, axis=1)
    iota = lax.broadcasted_iota(jnp.int32, (PB, size), 1)
    idx = jnp.min(jnp.where(g == rowmax[:, None], iota, size), axis=1)

    t = g - rowmax[:, None]
    e = jnp.exp(t)
    denom = jnp.sum(e, axis=1)
    num = jnp.sum(e * t, axis=1)
    kld = jnp.sum(num / denom - jnp.log(denom))

    onehot = (iota == idx[:, None]).astype(jnp.float32)
    zq = lax.dot_general(cb, onehot, (((0,), (1,)), ((), ())),
                         preferred_element_type=jnp.float32)
    zq_ref[0] = zq
    commit = w * jnp.sum((zb - zq) ** 2)

    @pl.when((b == 0) & (p == 0))
    def _():
        loss_ref[0, 0] = 0.0

    loss_ref[0, 0] += (kld + commit) * inv_bs


def kernel(z, codebook, var_q, var_init):
    bs, dim_z, d1, d2 = z.shape
    size, _ = codebook.shape
    npix = d1 * d2
    z3 = z.reshape(bs, dim_z, npix)

    var_q_eff = jax.nn.sigmoid(var_q) * 2.0 * var_init
    w = (0.5 / jnp.clip(var_q_eff, 1e-10, None)).reshape(1, 1)

    body = functools.partial(_vq_body, nb=bs, size=size, inv_bs=1.0 / bs)
    zq3, loss = pl.pallas_call(
        body,
        grid=(bs, npix // PB),
        in_specs=[
            pl.BlockSpec(memory_space=pltpu.SMEM),
            pl.BlockSpec((1, dim_z, PB), lambda b, p: (b, 0, p)),
            pl.BlockSpec((size, dim_z), lambda b, p: (0, 0)),
        ],
        out_specs=[
            pl.BlockSpec((1, dim_z, PB), lambda b, p: (b, 0, p)),
            pl.BlockSpec(memory_space=pltpu.SMEM),
        ],
        out_shape=[
            jax.ShapeDtypeStruct((bs, dim_z, npix), jnp.float32),
            jax.ShapeDtypeStruct((1, 1), jnp.float32),
        ],
    )(w, z3, codebook)
    return zq3.reshape(bs, dim_z, d1, d2), loss[0, 0]
